# baseline (speedup 1.0000x reference)
import functools

import jax
import jax.numpy as jnp
from jax import lax
from jax.experimental import pallas as pl
from jax.experimental.pallas import tpu as pltpu
from jax.experimental.pallas import tpu_sc as plsc

NC = 2
NS = 16
NW = NC * NS
L = 16

CAPW = 24576
CE = 64
CHK = 3200

H = 8
F = 32
HF = H * F


def _mesh():
  return plsc.VectorSubcoreMesh(
      core_axis_name="c", subcore_axis_name="s", num_cores=NC, num_subcores=NS
  )


def _bucketize(src, dst, npad):
  e = src.shape[0]
  nloc = npad // NW

  @functools.partial(
      pl.kernel,
      mesh=_mesh(),
      compiler_params=pltpu.CompilerParams(needs_layout_passes=False, use_tc_tiling_on_sc=False),
      out_type=[
          jax.ShapeDtypeStruct((NW * CAPW * 2,), jnp.int32),
          jax.ShapeDtypeStruct((NW * L,), jnp.int32),
      ],
      scratch_types=[
          pltpu.VMEM((CHK,), jnp.int32),
          pltpu.VMEM((CHK,), jnp.int32),
          pltpu.VMEM((CHK,), jnp.int32),
          pltpu.VMEM((CHK,), jnp.int32),
          pltpu.VMEM((CAPW * 2,), jnp.int32),
          pltpu.VMEM((L,), jnp.int32),
          pltpu.SemaphoreType.DMA,
          pltpu.SemaphoreType.DMA,
          pltpu.SemaphoreType.DMA,
          pltpu.SemaphoreType.DMA,
      ],
  )
  def k(src_hbm, dst_hbm, lists_hbm, cnt_hbm, sv0, dv0, sv1, dv1, cbuf, cw,
        sems0, semd0, sems1, semd1):
    wid = lax.axis_index("s") * NC + lax.axis_index("c")
    lo = wid * nloc
    zero_i = jnp.zeros((L,), jnp.int32)
    dummy_d = jnp.full((L,), nloc, jnp.int32)

    def mz(i, carry):
      base = i * 128
      for u in range(4):
        cbuf[pl.ds(base + u * L, L)] = zero_i
      for u in range(4):
        cbuf[pl.ds(base + 64 + u * L, L)] = dummy_d
      return carry

    lax.fori_loop(0, CAPW * 2 // 128, mz, 0)

    lane15 = jnp.full((L,), L - 1, jnp.int32)
    nchk = e // CHK

    def issue(ci, sv, dv, sems, semd):
      base = pl.multiple_of(ci * CHK, CHK)
      pltpu.async_copy(src_hbm.at[pl.ds(base, CHK)], sv, sems)
      pltpu.async_copy(dst_hbm.at[pl.ds(base, CHK)], dv, semd)

    def scan(sv, dv, sems, semd, offv):
      pltpu.make_async_copy(src_hbm.at[pl.ds(0, CHK)], sv, sems).wait()
      pltpu.make_async_copy(dst_hbm.at[pl.ds(0, CHK)], dv, semd).wait()

      def vec8(t, offv):
        svals, dvals, masks, csums, pcs = [], [], [], [], []
        for u in range(8):
          at = pl.ds((t * 8 + u) * L, L)
          d = dv[at]
          s = sv[at]
          m = (d >= lo) & (d < lo + nloc)
          c = plsc.cumsum(jnp.where(m, 1, 0))
          pcs.append(jnp.take_along_axis(c, lane15, axis=0,
                                         mode="promise_in_bounds"))
          svals.append(s)
          dvals.append(d - lo)
          masks.append(m)
          csums.append(c)
        for u in range(8):
          pos = offv + csums[u] - 1
          sidx = ((pos >> 6) << 7) + (pos & 63)
          plsc.store_scatter(cbuf, [sidx], svals[u], mask=masks[u])
          plsc.store_scatter(cbuf, [sidx + 64], dvals[u], mask=masks[u])
          offv = offv + pcs[u]
        return offv

      return lax.fori_loop(0, CHK // L // 8, vec8, offv)

    issue(0, sv0, dv0, sems0, semd0)

    def cpair(c2, offv):
      ci = c2 * 2

      @pl.when(ci + 1 < nchk)
      def _p1():
        issue(ci + 1, sv1, dv1, sems1, semd1)

      offv = scan(sv0, dv0, sems0, semd0, offv)

      @pl.when(ci + 2 < nchk)
      def _p0():
        issue(ci + 2, sv0, dv0, sems0, semd0)

      def s1(offv):
        return scan(sv1, dv1, sems1, semd1, offv)

      offv = lax.cond(ci + 1 < nchk, s1, lambda o: o, offv)
      return offv

    offv = lax.fori_loop(0, (nchk + 1) // 2, cpair, jnp.zeros((L,), jnp.int32))
    cw[...] = offv
    wbase = pl.multiple_of(wid * CAPW * 2, CAPW * 2)
    pltpu.sync_copy(cbuf, lists_hbm.at[pl.ds(wbase, CAPW * 2)])
    pltpu.sync_copy(cw, cnt_hbm.at[pl.ds(pl.multiple_of(wid * L, L), L)])

  return k(src, dst)


def _sc_aggregate(lists, cnt, feat, elr):
  npad = feat.shape[0]
  nloc = npad // NW
  den_sz = ((nloc + 1) * H + L - 1) // L * L

  @functools.partial(
      pl.kernel,
      mesh=_mesh(),
      compiler_params=pltpu.CompilerParams(needs_layout_passes=False, use_tc_tiling_on_sc=False),
      out_type=[
          jax.ShapeDtypeStruct((npad * HF,), jnp.float32),
          jax.ShapeDtypeStruct((npad * H,), jnp.float32),
      ],
      scratch_types=[
          pltpu.VMEM(((nloc + 1) * HF,), jnp.float32),
          pltpu.VMEM((den_sz,), jnp.float32),
          pltpu.VMEM((nloc + 1, 2 * H), jnp.float32),
          pltpu.VMEM((CE,), jnp.int32),
          pltpu.VMEM((CE,), jnp.int32),
          pltpu.VMEM((CE, 2 * H), jnp.float32),
          pltpu.VMEM((CE, HF // 2), jnp.int32),
          pltpu.VMEM((CE,), jnp.int32),
          pltpu.VMEM((CE,), jnp.int32),
          pltpu.VMEM((CE, 2 * H), jnp.float32),
          pltpu.VMEM((CE, HF // 2), jnp.int32),
          pltpu.VMEM((L,), jnp.int32),
          pltpu.SemaphoreType.DMA,
          pltpu.SemaphoreType.DMA,
          pltpu.SemaphoreType.DMA,
          pltpu.SemaphoreType.DMA,
      ],
  )
  def k(lists_hbm, cnt_hbm, feat_hbm, elr_hbm, acc_hbm, den_hbm,
        acc_l, den_l, elr_loc, sl0, dl0, elr_rows0, feat_rows0,
        sl1, dl1, elr_rows1, feat_rows1, cntv, semf0, seme0, semf1, seme1):
    wid = lax.axis_index("s") * NC + lax.axis_index("c")
    lo = wid * nloc
    iota = lax.iota(jnp.int32, L)
    low3 = iota & 7
    ge8 = (iota >= 8).astype(jnp.int32)
    zf = jnp.zeros((L,), jnp.float32)

    def zrow(i, carry):
      acc_l[pl.ds(i * L, L)] = zf
      return carry

    lax.fori_loop(0, (nloc + 1) * HF // L, zrow, 0)

    def zden(i, carry):
      den_l[pl.ds(i * L, L)] = zf
      return carry

    lax.fori_loop(0, den_sz // L, zden, 0)

    pltpu.sync_copy(elr_hbm.at[pl.ds(lo, nloc)], elr_loc.at[pl.ds(0, nloc)])
    elr_loc[nloc, pl.ds(0, L)] = zf

    pltpu.sync_copy(cnt_hbm.at[pl.ds(pl.multiple_of(wid * L, L), L)], cntv)
    cnt = jnp.max(cntv[...], axis=0)
    nch = (cnt + CE - 1) // CE
    wbase = pl.multiple_of(wid * CAPW * 2, CAPW * 2)

    def issue(kc, slb, dlb, fb, eb, semf, seme):
      base = pl.multiple_of(kc * 2 * CE, 2 * CE)
      pltpu.sync_copy(lists_hbm.at[pl.ds(wbase + base, CE)], slb)
      pltpu.sync_copy(lists_hbm.at[pl.ds(wbase + base + CE, CE)], dlb)
      pltpu.async_copy(feat_hbm.at[slb], fb, semf)
      pltpu.async_copy(elr_hbm.at[slb], eb, seme)

    def compute(slb, dlb, fb, eb, semf, seme):
      pltpu.make_async_copy(feat_hbm.at[slb], fb, semf).wait()
      pltpu.make_async_copy(elr_hbm.at[slb], eb, seme).wait()

      def group(j, carry2):
        dvec = dlb[pl.ds(j * L, L)]
        d2s, ee2s = [], []
        for p in range(8):
          rowpair = j * L + 2 * p + ge8
          el2 = plsc.load_gather(eb, [rowpair, low3])
          psel = jnp.full((L,), 2 * p, jnp.int32) + ge8
          d2 = jnp.take_along_axis(dvec, psel, axis=0,
                                   mode="promise_in_bounds")
          er2 = plsc.load_gather(elr_loc, [d2, low3 + 8])
          t = el2 + er2
          t = jnp.where(t >= 0, t, t * jnp.float32(0.2))
          ee2s.append(jnp.exp(t))
          d2s.append(d2)
        for p in range(8):
          plsc.addupdate_scatter(den_l, [d2s[p] * 8 + low3], ee2s[p])
        for p in range(8):
          ee2 = ee2s[p]
          for q in range(2):
            est = 2 * p + q
            dq = jnp.take_along_axis(dvec, jnp.full((L,), est, jnp.int32),
                                     axis=0, mode="promise_in_bounds")
            dqb = dq * HF
            ss = [
                jnp.take_along_axis(ee2, jnp.full((L,), 8 * q + h, jnp.int32),
                                    axis=0, mode="promise_in_bounds")
                for h in range(H)
            ]
            ws = [fb[j * L + est, pl.ds(h * L, L)] for h in range(H)]
            vals = []
            for h in range(H):
              lo = plsc.bitcast(ws[h] << 16, jnp.float32)
              hi = plsc.bitcast(ws[h] & jnp.int32(-65536), jnp.float32)
              vals.append(lo * ss[h])
              vals.append(hi * ss[h])
            for h in range(H):
              for half in range(2):
                col = h * F + half * L
                plsc.addupdate_scatter(acc_l, [dqb + (col + iota)],
                                       vals[2 * h + half])
        return carry2

      plsc.parallel_loop(0, CE // L, 1, unroll=2, carry=jnp.int32(0))(group)

    @pl.when(nch > 0)
    def _prime():
      issue(0, sl0, dl0, feat_rows0, elr_rows0, semf0, seme0)

    def pair(k2, carry):
      kc = k2 * 2

      @pl.when(kc + 1 < nch)
      def _i1():
        issue(kc + 1, sl1, dl1, feat_rows1, elr_rows1, semf1, seme1)

      compute(sl0, dl0, feat_rows0, elr_rows0, semf0, seme0)

      @pl.when(kc + 2 < nch)
      def _i0():
        issue(kc + 2, sl0, dl0, feat_rows0, elr_rows0, semf0, seme0)

      @pl.when(kc + 1 < nch)
      def _c1():
        compute(sl1, dl1, feat_rows1, elr_rows1, semf1, seme1)

      return carry

    lax.fori_loop(0, (nch + 1) // 2, pair, 0)

    pltpu.sync_copy(
        acc_l.at[pl.ds(0, nloc * HF)],
        acc_hbm.at[pl.ds(pl.multiple_of(lo * HF, HF), nloc * HF)])
    pltpu.sync_copy(
        den_l.at[pl.ds(0, nloc * H)],
        den_hbm.at[pl.ds(pl.multiple_of(lo * H, H), nloc * H)],
    )

  return k(lists, cnt, feat, elr)


BN = 512


def _gelu(o):
  return 0.5 * o * (1.0 + lax.erf(o * jnp.float32(0.7071067811865476)))


def _tc_pre_body(x_ref, w_ref, alr_ref, feat_ref, elr_ref):
  f = jnp.dot(x_ref[...], w_ref[...], preferred_element_type=jnp.float32)
  feat_ref[...] = f.astype(jnp.bfloat16)
  elr_ref[...] = jnp.dot(f, alr_ref[...], preferred_element_type=jnp.float32)


def _tc_pre(xpad, w, alr):
  npad = xpad.shape[0]
  grid = (npad // BN,)
  return pl.pallas_call(
      _tc_pre_body,
      grid=grid,
      in_specs=[
          pl.BlockSpec((BN, xpad.shape[1]), lambda i: (i, 0)),
          pl.BlockSpec(w.shape, lambda i: (0, 0)),
          pl.BlockSpec(alr.shape, lambda i: (0, 0)),
      ],
      out_specs=[
          pl.BlockSpec((BN, HF), lambda i: (i, 0)),
          pl.BlockSpec((BN, 2 * H), lambda i: (i, 0)),
      ],
      out_shape=[
          jax.ShapeDtypeStruct((npad, HF), jnp.bfloat16),
          jax.ShapeDtypeStruct((npad, 2 * H), jnp.float32),
      ],
  )(xpad, w, alr)


def _tc_mid_body(acc_ref, den_ref, b_ref, b8_ref, m8_ref, w_ref, alr_ref,
                 feat_ref, elr_ref):
  denb = jnp.dot(den_ref[...], b8_ref[...], preferred_element_type=jnp.float32)
  o = acc_ref[...] / (denb + jnp.float32(1e-9)) + b_ref[...]
  g = _gelu(o)
  xn = jnp.dot(g, m8_ref[...], preferred_element_type=jnp.float32)
  f = jnp.dot(xn, w_ref[...], preferred_element_type=jnp.float32)
  feat_ref[...] = f.astype(jnp.bfloat16)
  elr_ref[...] = jnp.dot(f, alr_ref[...], preferred_element_type=jnp.float32)


def _tc_mid(acc, den, bflat, b8, m8, w, alr):
  npad = acc.shape[0]
  grid = (npad // BN,)
  return pl.pallas_call(
      _tc_mid_body,
      grid=grid,
      in_specs=[
          pl.BlockSpec((BN, HF), lambda i: (i, 0)),
          pl.BlockSpec((BN, H), lambda i: (i, 0)),
          pl.BlockSpec((1, HF), lambda i: (0, 0)),
          pl.BlockSpec(b8.shape, lambda i: (0, 0)),
          pl.BlockSpec(m8.shape, lambda i: (0, 0)),
          pl.BlockSpec(w.shape, lambda i: (0, 0)),
          pl.BlockSpec(alr.shape, lambda i: (0, 0)),
      ],
      out_specs=[
          pl.BlockSpec((BN, HF), lambda i: (i, 0)),
          pl.BlockSpec((BN, 2 * H), lambda i: (i, 0)),
      ],
      out_shape=[
          jax.ShapeDtypeStruct((npad, HF), jnp.bfloat16),
          jax.ShapeDtypeStruct((npad, 2 * H), jnp.float32),
      ],
  )(acc, den, bflat, b8, m8, w, alr)


def _tc_fin_body(acc_ref, den_ref, b_ref, b8_ref, pt_ref, out_ref):
  denb = jnp.dot(den_ref[...], b8_ref[...], preferred_element_type=jnp.float32)
  o = acc_ref[...] / (denb + jnp.float32(1e-9)) + b_ref[...]
  out_ref[...] = jnp.dot(o, pt_ref[...], preferred_element_type=jnp.float32)


def _tc_fin(acc, den, bflat, b8, pt):
  npad = acc.shape[0]
  grid = (npad // BN,)
  return pl.pallas_call(
      _tc_fin_body,
      grid=grid,
      in_specs=[
          pl.BlockSpec((BN, HF), lambda i: (i, 0)),
          pl.BlockSpec((BN, H), lambda i: (i, 0)),
          pl.BlockSpec((1, HF), lambda i: (0, 0)),
          pl.BlockSpec(b8.shape, lambda i: (0, 0)),
          pl.BlockSpec(pt.shape, lambda i: (0, 0)),
      ],
      out_specs=pl.BlockSpec((BN, HF), lambda i: (i, 0)),
      out_shape=jax.ShapeDtypeStruct((npad, HF), jnp.float32),
  )(acc, den, bflat, b8, pt)


def _alr(al, ar):
  h, f = al.shape
  eye = jnp.eye(h, dtype=al.dtype)
  a = jnp.einsum("hf,hg->hfg", al, eye).reshape(h * f, h)
  r = jnp.einsum("hf,hg->hfg", ar, eye).reshape(h * f, h)
  return jnp.concatenate([a, r], axis=-1)


def kernel(x, edge_index, W1, al1, ar1, b1, W2, al2, ar2, b2,
           W3, al3, ar3, b3):
  n = x.shape[0]
  npad = (n + BN - 1) // BN * BN
  assert npad % BN == 0 and npad % NW == 0
  src = edge_index[0]
  dst = edge_index[1]
  xpad = jnp.pad(x, ((0, npad - n), (0, 0)))

  b8 = jnp.kron(jnp.eye(H, dtype=jnp.float32), jnp.ones((1, F), jnp.float32))
  m8 = jnp.kron(jnp.ones((H, 1), jnp.float32),
                jnp.eye(F, dtype=jnp.float32)) / H
  pp = jnp.arange(HF)
  sem = (pp // F) * F + (pp % F % L) * 2 + (pp % F) // L
  m8p = m8[sem]
  pt = jnp.zeros((HF, HF), jnp.float32).at[pp, sem].set(1.0)

  def permb(b):
    return b.reshape(-1)[sem].reshape(1, HF)

  def pack(feat_bf16):
    return lax.bitcast_convert_type(
        feat_bf16.reshape(npad, HF // 2, 2), jnp.int32)

  lists, cnt = _bucketize(src, dst, npad)

  feat, elr = _tc_pre(xpad, W1, _alr(al1, ar1))
  acc, den = _sc_aggregate(lists, cnt, pack(feat), elr)
  feat, elr = _tc_mid(acc.reshape(npad, HF), den.reshape(npad, H),
                      permb(b1), b8, m8p, W2, _alr(al2, ar2))
  acc, den = _sc_aggregate(lists, cnt, pack(feat), elr)
  feat, elr = _tc_mid(acc.reshape(npad, HF), den.reshape(npad, H),
                      permb(b2), b8, m8p, W3, _alr(al3, ar3))
  acc, den = _sc_aggregate(lists, cnt, pack(feat), elr)
  out = _tc_fin(acc.reshape(npad, HF), den.reshape(npad, H),
                permb(b3), b8, pt)
  return out[:n].reshape(n, H, F)


# speedup vs baseline: 1.0793x; 1.0793x over previous
import functools

import jax
import jax.numpy as jnp
from jax import lax
from jax.experimental import pallas as pl
from jax.experimental.pallas import tpu as pltpu
from jax.experimental.pallas import tpu_sc as plsc

NC = 2
NS = 16
NW = NC * NS
L = 16

CAPW = 24576
CE = 64
CHK = 3200

H = 8
F = 32
HF = H * F


def _mesh():
  return plsc.VectorSubcoreMesh(
      core_axis_name="c", subcore_axis_name="s", num_cores=NC, num_subcores=NS
  )


def _bucketize(src, dst, npad):
  e = src.shape[0]
  nloc = npad // NW

  @functools.partial(
      pl.kernel,
      mesh=_mesh(),
      compiler_params=pltpu.CompilerParams(needs_layout_passes=False, use_tc_tiling_on_sc=False),
      out_type=[
          jax.ShapeDtypeStruct((NW * CAPW * 2,), jnp.int32),
          jax.ShapeDtypeStruct((NW * L,), jnp.int32),
      ],
      scratch_types=[
          pltpu.VMEM((CHK,), jnp.int32),
          pltpu.VMEM((CHK,), jnp.int32),
          pltpu.VMEM((CHK,), jnp.int32),
          pltpu.VMEM((CHK,), jnp.int32),
          pltpu.VMEM((CAPW * 2,), jnp.int32),
          pltpu.VMEM((L,), jnp.int32),
          pltpu.SemaphoreType.DMA,
          pltpu.SemaphoreType.DMA,
          pltpu.SemaphoreType.DMA,
          pltpu.SemaphoreType.DMA,
      ],
  )
  def k(src_hbm, dst_hbm, lists_hbm, cnt_hbm, sv0, dv0, sv1, dv1, cbuf, cw,
        sems0, semd0, sems1, semd1):
    wid = lax.axis_index("s") * NC + lax.axis_index("c")
    lo = wid * nloc
    zero_i = jnp.zeros((L,), jnp.int32)
    dummy_d = jnp.full((L,), nloc, jnp.int32)

    def mz(i, carry):
      base = i * 128
      for u in range(4):
        cbuf[pl.ds(base + u * L, L)] = zero_i
      for u in range(4):
        cbuf[pl.ds(base + 64 + u * L, L)] = dummy_d
      return carry

    lax.fori_loop(0, CAPW * 2 // 128, mz, 0)

    lane15 = jnp.full((L,), L - 1, jnp.int32)
    nchk = e // CHK

    def issue(ci, sv, dv, sems, semd):
      base = pl.multiple_of(ci * CHK, CHK)
      pltpu.async_copy(src_hbm.at[pl.ds(base, CHK)], sv, sems)
      pltpu.async_copy(dst_hbm.at[pl.ds(base, CHK)], dv, semd)

    def scan(sv, dv, sems, semd, offv):
      pltpu.make_async_copy(src_hbm.at[pl.ds(0, CHK)], sv, sems).wait()
      pltpu.make_async_copy(dst_hbm.at[pl.ds(0, CHK)], dv, semd).wait()

      def vec8(t, offv):
        svals, dvals, masks, csums, pcs = [], [], [], [], []
        for u in range(8):
          at = pl.ds((t * 8 + u) * L, L)
          d = dv[at]
          s = sv[at]
          m = (d >= lo) & (d < lo + nloc)
          c = plsc.cumsum(jnp.where(m, 1, 0))
          pcs.append(jnp.take_along_axis(c, lane15, axis=0,
                                         mode="promise_in_bounds"))
          svals.append(s)
          dvals.append(d - lo)
          masks.append(m)
          csums.append(c)
        for u in range(8):
          pos = offv + csums[u] - 1
          sidx = ((pos >> 6) << 7) + (pos & 63)
          plsc.store_scatter(cbuf, [sidx], svals[u], mask=masks[u])
          plsc.store_scatter(cbuf, [sidx + 64], dvals[u], mask=masks[u])
          offv = offv + pcs[u]
        return offv

      return lax.fori_loop(0, CHK // L // 8, vec8, offv)

    issue(0, sv0, dv0, sems0, semd0)

    def cpair(c2, offv):
      ci = c2 * 2

      @pl.when(ci + 1 < nchk)
      def _p1():
        issue(ci + 1, sv1, dv1, sems1, semd1)

      offv = scan(sv0, dv0, sems0, semd0, offv)

      @pl.when(ci + 2 < nchk)
      def _p0():
        issue(ci + 2, sv0, dv0, sems0, semd0)

      def s1(offv):
        return scan(sv1, dv1, sems1, semd1, offv)

      offv = lax.cond(ci + 1 < nchk, s1, lambda o: o, offv)
      return offv

    offv = lax.fori_loop(0, (nchk + 1) // 2, cpair, jnp.zeros((L,), jnp.int32))
    cw[...] = offv
    wbase = pl.multiple_of(wid * CAPW * 2, CAPW * 2)
    pltpu.sync_copy(cbuf, lists_hbm.at[pl.ds(wbase, CAPW * 2)])
    pltpu.sync_copy(cw, cnt_hbm.at[pl.ds(pl.multiple_of(wid * L, L), L)])

  return k(src, dst)


def _sc_aggregate(lists, cnt, feat, elr):
  npad = feat.shape[0]
  nloc = npad // NW
  den_sz = ((nloc + 1) * H + L - 1) // L * L

  @functools.partial(
      pl.kernel,
      mesh=_mesh(),
      compiler_params=pltpu.CompilerParams(needs_layout_passes=False, use_tc_tiling_on_sc=False),
      out_type=[
          jax.ShapeDtypeStruct((npad * HF,), jnp.float32),
          jax.ShapeDtypeStruct((npad * H,), jnp.float32),
      ],
      scratch_types=[
          pltpu.VMEM(((nloc + 1) * HF,), jnp.float32),
          pltpu.VMEM((den_sz,), jnp.float32),
          pltpu.VMEM((nloc + 1, 2 * H), jnp.float32),
          pltpu.VMEM((CE,), jnp.int32),
          pltpu.VMEM((CE,), jnp.int32),
          pltpu.VMEM((CE, 2 * H), jnp.float32),
          pltpu.VMEM((CE, HF // 2), jnp.int32),
          pltpu.VMEM((CE,), jnp.int32),
          pltpu.VMEM((CE,), jnp.int32),
          pltpu.VMEM((CE, 2 * H), jnp.float32),
          pltpu.VMEM((CE, HF // 2), jnp.int32),
          pltpu.VMEM((L,), jnp.int32),
          pltpu.SemaphoreType.DMA,
          pltpu.SemaphoreType.DMA,
          pltpu.SemaphoreType.DMA,
          pltpu.SemaphoreType.DMA,
      ],
  )
  def k(lists_hbm, cnt_hbm, feat_hbm, elr_hbm, acc_hbm, den_hbm,
        acc_l, den_l, elr_loc, sl0, dl0, elr_rows0, feat_rows0,
        sl1, dl1, elr_rows1, feat_rows1, cntv, semf0, seme0, semf1, seme1):
    wid = lax.axis_index("s") * NC + lax.axis_index("c")
    lo = wid * nloc
    iota = lax.iota(jnp.int32, L)
    low3 = iota & 7
    ge8 = (iota >= 8).astype(jnp.int32)
    zf = jnp.zeros((L,), jnp.float32)

    def zrow(i, carry):
      acc_l[pl.ds(i * L, L)] = zf
      return carry

    lax.fori_loop(0, (nloc + 1) * HF // L, zrow, 0)

    def zden(i, carry):
      den_l[pl.ds(i * L, L)] = zf
      return carry

    lax.fori_loop(0, den_sz // L, zden, 0)

    pltpu.sync_copy(elr_hbm.at[pl.ds(lo, nloc)], elr_loc.at[pl.ds(0, nloc)])
    elr_loc[nloc, pl.ds(0, L)] = zf

    pltpu.sync_copy(cnt_hbm.at[pl.ds(pl.multiple_of(wid * L, L), L)], cntv)
    cnt = jnp.max(cntv[...], axis=0)
    nch = (cnt + CE - 1) // CE
    wbase = pl.multiple_of(wid * CAPW * 2, CAPW * 2)

    def issue(kc, slb, dlb, fb, eb, semf, seme):
      base = pl.multiple_of(kc * 2 * CE, 2 * CE)
      pltpu.sync_copy(lists_hbm.at[pl.ds(wbase + base, CE)], slb)
      pltpu.sync_copy(lists_hbm.at[pl.ds(wbase + base + CE, CE)], dlb)
      pltpu.async_copy(feat_hbm.at[slb], fb, semf)
      pltpu.async_copy(elr_hbm.at[slb], eb, seme)

    def compute(slb, dlb, fb, eb, semf, seme):
      pltpu.make_async_copy(feat_hbm.at[slb], fb, semf).wait()
      pltpu.make_async_copy(elr_hbm.at[slb], eb, seme).wait()

      def group(j, carry2):
        dvec = dlb[pl.ds(j * L, L)]
        d2s, ee2s = [], []
        for p in range(8):
          rowpair = j * L + 2 * p + ge8
          el2 = plsc.load_gather(eb, [rowpair, low3])
          psel = jnp.full((L,), 2 * p, jnp.int32) + ge8
          d2 = jnp.take_along_axis(dvec, psel, axis=0,
                                   mode="promise_in_bounds")
          er2 = plsc.load_gather(elr_loc, [d2, low3 + 8])
          t = el2 + er2
          t = jnp.where(t >= 0, t, t * jnp.float32(0.2))
          ee2s.append(jnp.exp(t))
          d2s.append(d2)
        for p in range(8):
          plsc.addupdate_scatter(den_l, [d2s[p] * 8 + low3], ee2s[p])
        for p in range(8):
          ee2 = ee2s[p]
          for q in range(2):
            est = 2 * p + q
            dq = jnp.take_along_axis(dvec, jnp.full((L,), est, jnp.int32),
                                     axis=0, mode="promise_in_bounds")
            dqb = dq * HF
            ss = [
                jnp.take_along_axis(ee2, jnp.full((L,), 8 * q + h, jnp.int32),
                                    axis=0, mode="promise_in_bounds")
                for h in range(H)
            ]
            ws = [fb[j * L + est, pl.ds(h * L, L)] for h in range(H)]
            vals = []
            for h in range(H):
              lo = plsc.bitcast(ws[h] << 16, jnp.float32)
              hi = plsc.bitcast(ws[h] & jnp.int32(-65536), jnp.float32)
              vals.append(lo * ss[h])
              vals.append(hi * ss[h])
            for h in range(H):
              for half in range(2):
                col = h * F + half * L
                plsc.addupdate_scatter(acc_l, [dqb + (col + iota)],
                                       vals[2 * h + half])
        return carry2

      lax.fori_loop(0, CE // L, group, 0)

    @pl.when(nch > 0)
    def _prime():
      issue(0, sl0, dl0, feat_rows0, elr_rows0, semf0, seme0)

    def pair(k2, carry):
      kc = k2 * 2

      @pl.when(kc + 1 < nch)
      def _i1():
        issue(kc + 1, sl1, dl1, feat_rows1, elr_rows1, semf1, seme1)

      compute(sl0, dl0, feat_rows0, elr_rows0, semf0, seme0)

      @pl.when(kc + 2 < nch)
      def _i0():
        issue(kc + 2, sl0, dl0, feat_rows0, elr_rows0, semf0, seme0)

      @pl.when(kc + 1 < nch)
      def _c1():
        compute(sl1, dl1, feat_rows1, elr_rows1, semf1, seme1)

      return carry

    lax.fori_loop(0, (nch + 1) // 2, pair, 0)

    pltpu.sync_copy(
        acc_l.at[pl.ds(0, nloc * HF)],
        acc_hbm.at[pl.ds(pl.multiple_of(lo * HF, HF), nloc * HF)])
    pltpu.sync_copy(
        den_l.at[pl.ds(0, nloc * H)],
        den_hbm.at[pl.ds(pl.multiple_of(lo * H, H), nloc * H)],
    )

  return k(lists, cnt, feat, elr)


BN = 512


def _gelu(o):
  return 0.5 * o * (1.0 + lax.erf(o * jnp.float32(0.7071067811865476)))


def _tc_pre_body(x_ref, w_ref, alr_ref, feat_ref, elr_ref):
  f = jnp.dot(x_ref[...], w_ref[...], preferred_element_type=jnp.float32)
  feat_ref[...] = f.astype(jnp.bfloat16)
  elr_ref[...] = jnp.dot(f, alr_ref[...], preferred_element_type=jnp.float32)


def _tc_pre(xpad, w, alr):
  npad = xpad.shape[0]
  grid = (npad // BN,)
  return pl.pallas_call(
      _tc_pre_body,
      grid=grid,
      in_specs=[
          pl.BlockSpec((BN, xpad.shape[1]), lambda i: (i, 0)),
          pl.BlockSpec(w.shape, lambda i: (0, 0)),
          pl.BlockSpec(alr.shape, lambda i: (0, 0)),
      ],
      out_specs=[
          pl.BlockSpec((BN, HF), lambda i: (i, 0)),
          pl.BlockSpec((BN, 2 * H), lambda i: (i, 0)),
      ],
      out_shape=[
          jax.ShapeDtypeStruct((npad, HF), jnp.bfloat16),
          jax.ShapeDtypeStruct((npad, 2 * H), jnp.float32),
      ],
  )(xpad, w, alr)


def _tc_mid_body(acc_ref, den_ref, b_ref, b8_ref, m8_ref, w_ref, alr_ref,
                 feat_ref, elr_ref):
  denb = jnp.dot(den_ref[...], b8_ref[...], preferred_element_type=jnp.float32)
  o = acc_ref[...] / (denb + jnp.float32(1e-9)) + b_ref[...]
  g = _gelu(o)
  xn = jnp.dot(g, m8_ref[...], preferred_element_type=jnp.float32)
  f = jnp.dot(xn, w_ref[...], preferred_element_type=jnp.float32)
  feat_ref[...] = f.astype(jnp.bfloat16)
  elr_ref[...] = jnp.dot(f, alr_ref[...], preferred_element_type=jnp.float32)


def _tc_mid(acc, den, bflat, b8, m8, w, alr):
  npad = acc.shape[0]
  grid = (npad // BN,)
  return pl.pallas_call(
      _tc_mid_body,
      grid=grid,
      in_specs=[
          pl.BlockSpec((BN, HF), lambda i: (i, 0)),
          pl.BlockSpec((BN, H), lambda i: (i, 0)),
          pl.BlockSpec((1, HF), lambda i: (0, 0)),
          pl.BlockSpec(b8.shape, lambda i: (0, 0)),
          pl.BlockSpec(m8.shape, lambda i: (0, 0)),
          pl.BlockSpec(w.shape, lambda i: (0, 0)),
          pl.BlockSpec(alr.shape, lambda i: (0, 0)),
      ],
      out_specs=[
          pl.BlockSpec((BN, HF), lambda i: (i, 0)),
          pl.BlockSpec((BN, 2 * H), lambda i: (i, 0)),
      ],
      out_shape=[
          jax.ShapeDtypeStruct((npad, HF), jnp.bfloat16),
          jax.ShapeDtypeStruct((npad, 2 * H), jnp.float32),
      ],
  )(acc, den, bflat, b8, m8, w, alr)


def _tc_fin_body(acc_ref, den_ref, b_ref, b8_ref, pt_ref, out_ref):
  denb = jnp.dot(den_ref[...], b8_ref[...], preferred_element_type=jnp.float32)
  o = acc_ref[...] / (denb + jnp.float32(1e-9)) + b_ref[...]
  out_ref[...] = jnp.dot(o, pt_ref[...], preferred_element_type=jnp.float32)


def _tc_fin(acc, den, bflat, b8, pt):
  npad = acc.shape[0]
  grid = (npad // BN,)
  return pl.pallas_call(
      _tc_fin_body,
      grid=grid,
      in_specs=[
          pl.BlockSpec((BN, HF), lambda i: (i, 0)),
          pl.BlockSpec((BN, H), lambda i: (i, 0)),
          pl.BlockSpec((1, HF), lambda i: (0, 0)),
          pl.BlockSpec(b8.shape, lambda i: (0, 0)),
          pl.BlockSpec(pt.shape, lambda i: (0, 0)),
      ],
      out_specs=pl.BlockSpec((BN, HF), lambda i: (i, 0)),
      out_shape=jax.ShapeDtypeStruct((npad, HF), jnp.float32),
  )(acc, den, bflat, b8, pt)


def _alr(al, ar):
  h, f = al.shape
  eye = jnp.eye(h, dtype=al.dtype)
  a = jnp.einsum("hf,hg->hfg", al, eye).reshape(h * f, h)
  r = jnp.einsum("hf,hg->hfg", ar, eye).reshape(h * f, h)
  return jnp.concatenate([a, r], axis=-1)


def kernel(x, edge_index, W1, al1, ar1, b1, W2, al2, ar2, b2,
           W3, al3, ar3, b3):
  n = x.shape[0]
  npad = (n + BN - 1) // BN * BN
  assert npad % BN == 0 and npad % NW == 0
  src = edge_index[0]
  dst = edge_index[1]
  xpad = jnp.pad(x, ((0, npad - n), (0, 0)))

  b8 = jnp.kron(jnp.eye(H, dtype=jnp.float32), jnp.ones((1, F), jnp.float32))
  m8 = jnp.kron(jnp.ones((H, 1), jnp.float32),
                jnp.eye(F, dtype=jnp.float32)) / H
  pp = jnp.arange(HF)
  sem = (pp // F) * F + (pp % F % L) * 2 + (pp % F) // L
  m8p = m8[sem]
  pt = jnp.zeros((HF, HF), jnp.float32).at[pp, sem].set(1.0)

  def permb(b):
    return b.reshape(-1)[sem].reshape(1, HF)

  def pack(feat_bf16):
    return lax.bitcast_convert_type(
        feat_bf16.reshape(npad, HF // 2, 2), jnp.int32)

  lists, cnt = _bucketize(src, dst, npad)

  feat, elr = _tc_pre(xpad, W1, _alr(al1, ar1))
  acc, den = _sc_aggregate(lists, cnt, pack(feat), elr)
  feat, elr = _tc_mid(acc.reshape(npad, HF), den.reshape(npad, H),
                      permb(b1), b8, m8p, W2, _alr(al2, ar2))
  acc, den = _sc_aggregate(lists, cnt, pack(feat), elr)
  feat, elr = _tc_mid(acc.reshape(npad, HF), den.reshape(npad, H),
                      permb(b2), b8, m8p, W3, _alr(al3, ar3))
  acc, den = _sc_aggregate(lists, cnt, pack(feat), elr)
  out = _tc_fin(acc.reshape(npad, HF), den.reshape(npad, H),
                permb(b3), b8, pt)
  return out[:n].reshape(n, H, F)
